# baseline (speedup 1.0000x reference)
import jax
import jax.numpy as jnp
from jax import lax
from jax.experimental import pallas as pl
from jax.experimental.pallas import tpu as pltpu
from jax.experimental.pallas import tpu_sc as plsc
from jax._src.pallas import mpmd as _plmpmd

R, C = 128, 32768
BC = 16384
NB = C // BC
INT_MAX = 2147483647

NC, NS = 2, 16
NW = NC * NS
ZROWS = 8
ZBUF = 2048
ZCOLS = C // 2
NZDMA = ZCOLS // ZBUF


def _zerofill_body(out_hbm, zbuf, sem):
    w = lax.axis_index("s") * NC + lax.axis_index("c")
    row0 = (w % 16) * ZROWS
    col0 = (w // 16) * ZCOLS

    def zero_buf(t, c):
        zbuf[t // (ZBUF // 16), pl.ds((t % (ZBUF // 16)) * 16, 16)] = (
            jnp.zeros((16,), jnp.float32)
        )
        return c

    lax.fori_loop(0, ZROWS * ZBUF // 16, zero_buf, 0)
    zbuf2d = zbuf

    def fire(k, c):
        pltpu.async_copy(
            zbuf2d,
            out_hbm.at[pl.ds(row0, ZROWS), pl.ds(col0 + k * ZBUF, ZBUF)],
            sem,
        ).start()
        return c

    lax.fori_loop(0, NZDMA, fire, 0)

    def drain(k, c):
        pltpu.async_copy(
            zbuf2d,
            out_hbm.at[pl.ds(row0, ZROWS), pl.ds(col0, ZBUF)],
            sem,
        ).wait()
        return c

    lax.fori_loop(0, NZDMA, drain, 0)


_sc_zerofill = _plmpmd._mpmd_map(
    [(
        plsc.VectorSubcoreMesh(core_axis_name="c", subcore_axis_name="s"),
        _zerofill_body,
    )],
    out_types=jax.ShapeDtypeStruct((R, C), jnp.float32),
    scratch_types=[
        pltpu.VMEM((ZROWS, ZBUF), jnp.float32),
        pltpu.SemaphoreType.DMA,
    ],
)


def _amax_body(x_ref, idx_ref, fix_ref, m_scr, i_scr):
    j = pl.program_id(0)
    blk = x_ref[...]
    m = jnp.max(blk, axis=1, keepdims=True)
    liota = lax.broadcasted_iota(jnp.int32, blk.shape, 1)
    cand = jnp.where(blk == m, liota, INT_MAX)
    ci = jnp.min(cand, axis=1, keepdims=True) + j * BC

    @pl.when(j == 0)
    def _():
        m_scr[...] = m
        i_scr[...] = ci

    @pl.when(j > 0)
    def _():
        upd = m > m_scr[...]
        i_scr[...] = jnp.where(upd, ci, i_scr[...])
        m_scr[...] = jnp.where(upd, m, m_scr[...])

    @pl.when(j == NB - 1)
    def _():
        idx_ref[...] = i_scr[...]
        lane = lax.broadcasted_iota(jnp.int32, (R, 128), 1)
        fix_ref[...] = jnp.where(
            lane == i_scr[...] % 128, 1.0, 0.0
        ).astype(jnp.float32)


def _fixup_body(idx_smem, fix_ref, zeros_ref, out_ref, fsem):
    del zeros_ref

    def fire(r, c):
        base = (idx_smem[r, 0] // 128) * 128
        pltpu.make_async_copy(
            fix_ref.at[pl.ds(r, 1), :],
            out_ref.at[pl.ds(r, 1), pl.ds(base, 128)],
            fsem,
        ).start()
        return c

    lax.fori_loop(0, R, fire, 0)

    def fdrain(_, c):
        pltpu.make_async_copy(
            fix_ref.at[pl.ds(0, 1), :],
            out_ref.at[pl.ds(0, 1), pl.ds(0, 128)],
            fsem,
        ).wait()
        return c

    lax.fori_loop(0, R, fdrain, 0)


def kernel(x):
    zeros2d = _sc_zerofill()

    idx, fixline = pl.pallas_call(
        _amax_body,
        grid=(NB,),
        in_specs=[pl.BlockSpec((R, BC), lambda j: (0, j))],
        out_specs=[
            pl.BlockSpec((R, 1), lambda j: (0, 0)),
            pl.BlockSpec((R, 128), lambda j: (0, 0)),
        ],
        out_shape=[
            jax.ShapeDtypeStruct((R, 1), jnp.int32),
            jax.ShapeDtypeStruct((R, 128), jnp.float32),
        ],
        scratch_shapes=[
            pltpu.VMEM((R, 1), jnp.float32),
            pltpu.VMEM((R, 1), jnp.int32),
        ],
        compiler_params=pltpu.CompilerParams(
            dimension_semantics=("arbitrary",),
        ),
    )(x)

    return pl.pallas_call(
        _fixup_body,
        grid=(1,),
        in_specs=[
            pl.BlockSpec(memory_space=pltpu.SMEM),
            pl.BlockSpec((R, 128), lambda i: (0, 0)),
            pl.BlockSpec(memory_space=pl.ANY),
        ],
        out_specs=pl.BlockSpec(memory_space=pl.ANY),
        out_shape=jax.ShapeDtypeStruct((R, C), jnp.float32),
        input_output_aliases={2: 0},
        scratch_shapes=[pltpu.SemaphoreType.DMA],
    )(idx, fixline, zeros2d)


# speedup vs baseline: 3.0042x; 3.0042x over previous
import jax
import jax.numpy as jnp
from jax import lax
from jax.experimental import pallas as pl
from jax.experimental.pallas import tpu as pltpu

R, C = 128, 32768
BC = 16384
NB = C // BC
RH = R // 2
INT_MAX = 2147483647


def _body(x0_ref, x1_ref, out_ref, m_scr, i_scr, zsc, fix_scr, ismem,
          zsem, isem, fsem):
    j = pl.program_id(0)
    liota = lax.broadcasted_iota(jnp.int32, (RH, BC), 1)
    ms, cis = [], []
    for xr in (x0_ref, x1_ref):
        blk = xr[...]
        m = jnp.max(blk, axis=1, keepdims=True)
        cand = jnp.where(blk == m, liota, INT_MAX)
        ci = jnp.min(cand, axis=1, keepdims=True) + j * BC
        ms.append(m)
        cis.append(ci)
    m = jnp.concatenate(ms, axis=0)
    ci = jnp.concatenate(cis, axis=0)

    @pl.when(j == 0)
    def _():
        m_scr[...] = m
        i_scr[...] = ci
        zsc[...] = jnp.zeros((R, BC), jnp.float32)

    @pl.when(j > 0)
    def _():
        upd = m > m_scr[...]
        i_scr[...] = jnp.where(upd, ci, i_scr[...])
        m_scr[...] = jnp.where(upd, m, m_scr[...])

    @pl.when(j == 0)
    def _():
        for jb in range(NB):
            pltpu.make_async_copy(
                zsc.at[pl.ds(0, RH), :],
                out_ref.at[pl.ds(0, RH), pl.ds(jb * BC, BC)],
                zsem,
            ).start()
            pltpu.make_async_copy(
                zsc.at[pl.ds(RH, RH), :],
                out_ref.at[pl.ds(RH, RH), pl.ds(jb * BC, BC)],
                zsem,
            ).start()

    @pl.when(j == NB - 1)
    def _():
        pltpu.make_async_copy(i_scr, ismem, isem).start()
        lane = lax.broadcasted_iota(jnp.int32, (R, 128), 1)
        fix_scr[...] = jnp.where(
            lane == i_scr[...] % 128, 1.0, 0.0
        ).astype(jnp.float32)

        def zdrain(_, c):
            pltpu.make_async_copy(
                zsc.at[pl.ds(0, RH), :],
                out_ref.at[pl.ds(0, RH), pl.ds(0, BC)],
                zsem,
            ).wait()
            return c

        lax.fori_loop(0, 2 * NB, zdrain, 0)
        pltpu.make_async_copy(i_scr, ismem, isem).wait()

        def fire(r, c):
            base = (ismem[r, 0] // 128) * 128
            pltpu.make_async_copy(
                fix_scr.at[pl.ds(r, 1), :],
                out_ref.at[pl.ds(r, 1), pl.ds(base, 128)],
                fsem,
            ).start()
            return c

        lax.fori_loop(0, R, fire, 0)

        def fdrain(_, c):
            pltpu.make_async_copy(
                fix_scr.at[pl.ds(0, 1), :],
                out_ref.at[pl.ds(0, 1), pl.ds(0, 128)],
                fsem,
            ).wait()
            return c

        lax.fori_loop(0, R, fdrain, 0)


def kernel(x):
    return pl.pallas_call(
        _body,
        grid=(NB,),
        in_specs=[
            pl.BlockSpec((RH, BC), lambda j: (0, j)),
            pl.BlockSpec((RH, BC), lambda j: (1, j)),
        ],
        out_specs=pl.BlockSpec(memory_space=pl.ANY),
        out_shape=jax.ShapeDtypeStruct((R, C), jnp.float32),
        scratch_shapes=[
            pltpu.VMEM((R, 1), jnp.float32),
            pltpu.VMEM((R, 1), jnp.int32),
            pltpu.VMEM((R, BC), jnp.float32),
            pltpu.VMEM((R, 128), jnp.float32),
            pltpu.SMEM((R, 1), jnp.int32),
            pltpu.SemaphoreType.DMA,
            pltpu.SemaphoreType.DMA,
            pltpu.SemaphoreType.DMA,
        ],
        compiler_params=pltpu.CompilerParams(
            dimension_semantics=("arbitrary",),
        ),
    )(x, x)
